# baseline (speedup 1.0000x reference)
import functools

import jax
import jax.numpy as jnp
from jax import lax
from jax.experimental import pallas as pl
from jax.experimental.pallas import tpu as pltpu
from jax.experimental.pallas import tpu_sc as plsc

_NUM_CORES = 2
_NUM_SUBCORES = 16
_NUM_WORKERS = _NUM_CORES * _NUM_SUBCORES


@functools.lru_cache(maxsize=None)
def _broadcast_kernel(batch, seq, hidden):
    rows_per_worker = seq // _NUM_WORKERS
    chunk = min(rows_per_worker, 32)
    num_chunks = rows_per_worker // chunk
    mesh = plsc.VectorSubcoreMesh(core_axis_name="c", subcore_axis_name="s")

    @functools.partial(
        pl.kernel,
        mesh=mesh,
        out_type=jax.ShapeDtypeStruct((batch, seq, hidden), jnp.float32),
        scratch_types=[
            pltpu.VMEM((chunk, hidden), jnp.float32),
            pltpu.VMEM((chunk, hidden), jnp.float32),
            pltpu.SemaphoreType.DMA,
            pltpu.SemaphoreType.DMA,
            pltpu.SemaphoreType.DMA,
        ],
    )
    def k(table_hbm, out_hbm, buf0, buf1, ld, st0, st1):
        wid = lax.axis_index("s") * _NUM_CORES + lax.axis_index("c")
        base = wid * rows_per_worker
        bufs = (buf0, buf1)
        sts = (st0, st1)
        loads = [None] * num_chunks
        stores = [None] * num_chunks
        loads[0] = pltpu.async_copy(
            table_hbm.at[pl.ds(base, chunk), :], buf0, ld)
        for i in range(num_chunks):
            buf = bufs[i % 2]
            loads[i].wait()
            if i + 1 < num_chunks:
                if i >= 1:
                    for h in stores[i - 1]:
                        h.wait()
                row_n = base + (i + 1) * chunk
                loads[i + 1] = pltpu.async_copy(
                    table_hbm.at[pl.ds(row_n, chunk), :],
                    bufs[(i + 1) % 2], ld)
            row0 = base + i * chunk
            stores[i] = [
                pltpu.async_copy(
                    buf, out_hbm.at[b, pl.ds(row0, chunk), :], sts[i % 2])
                for b in range(batch)
            ]
        for i in (num_chunks - 2, num_chunks - 1):
            if i >= 0:
                for h in stores[i]:
                    h.wait()

    return k


def kernel(x, pos_table):
    batch = x.shape[0]
    seq, hidden = pos_table.shape
    return _broadcast_kernel(batch, seq, hidden)(pos_table)
